# baseline (speedup 1.0000x reference)
import functools

import jax
import jax.numpy as jnp
import numpy as np
from jax.experimental import pallas as pl
from jax.experimental.pallas import tpu as pltpu
from jax.sharding import Mesh, PartitionSpec as P

try:
    from jax import shard_map as _shard_map
except ImportError:
    from jax.experimental.shard_map import shard_map as _shard_map

_TILE = 512


def _dot_tb(a, b):
    return jax.lax.dot_general(a, b, (((1,), (1,)), ((), ())),
                               preferred_element_type=jnp.float32)


def _dot_ta(a, w):
    return jax.lax.dot_general(a, w, (((0,), (0,)), ((), ())),
                               preferred_element_type=jnp.float32)


def _sweep_kernel(xT_ref, L_ref, t1_ref, z_ref):
    j = pl.program_id(0)

    y = _dot_tb(xT_ref[...], L_ref[...])
    t1_ref[...] = y

    @pl.when(j == 0)
    def _init():
        z_ref[...] = jnp.zeros_like(z_ref)

    z_ref[...] += jax.lax.dot_general(y, L_ref[...], (((1,), (0,)), ((), ())),
                                      preferred_element_type=jnp.float32)


def _combine_kernel(x_ref, t1T_ref, z_ref, w_ref, o_ref):
    hi = jax.lax.Precision.HIGHEST
    out = jnp.dot(x_ref[...], w_ref[0], preferred_element_type=jnp.float32,
                  precision=hi)
    out += jax.lax.dot_general(t1T_ref[...], w_ref[1], (((0,), (0,)), ((), ())),
                               preferred_element_type=jnp.float32, precision=hi)
    out += jax.lax.dot_general(z_ref[0] + z_ref[1], 2.0 * w_ref[2],
                               (((0,), (0,)), ((), ())),
                               preferred_element_type=jnp.float32, precision=hi)
    o_ref[...] = out


def _run_local(xT, x_l, L_l, wmod, *, tile, n, in_f, out_f):
    nl = L_l.shape[0]
    ntl = nl // tile

    t1T_l, z_l = pl.pallas_call(
        _sweep_kernel,
        out_shape=(jax.ShapeDtypeStruct((in_f, nl), jnp.float32),
                   jax.ShapeDtypeStruct((in_f, n), jnp.float32)),
        grid=(ntl,),
        in_specs=[
            pl.BlockSpec((in_f, n), lambda j: (0, 0)),
            pl.BlockSpec((tile, n), lambda j: (j, 0)),
        ],
        out_specs=(
            pl.BlockSpec((in_f, tile), lambda j: (0, j)),
            pl.BlockSpec((in_f, n), lambda j: (0, 0)),
        ),
        compiler_params=pltpu.CompilerParams(
            dimension_semantics=("arbitrary",),
            vmem_limit_bytes=48 * 1024 * 1024),
    )(xT, L_l)

    zall = jax.lax.all_gather(z_l, "d")
    lo = jax.lax.axis_index("d") * nl
    zloc = jax.lax.dynamic_slice(zall, (0, 0, lo), (2, in_f, nl))

    out_l = pl.pallas_call(
        _combine_kernel,
        out_shape=jax.ShapeDtypeStruct((nl, out_f), jnp.float32),
        grid=(ntl,),
        in_specs=[
            pl.BlockSpec((tile, in_f), lambda i: (i, 0)),
            pl.BlockSpec((in_f, tile), lambda i: (0, i)),
            pl.BlockSpec((2, in_f, tile), lambda i: (0, 0, i)),
            pl.BlockSpec((3, in_f, out_f), lambda i: (0, 0, 0)),
        ],
        out_specs=pl.BlockSpec((tile, out_f), lambda i: (i, 0)),
        compiler_params=pltpu.CompilerParams(
            dimension_semantics=("arbitrary",),
            vmem_limit_bytes=32 * 1024 * 1024),
    )(x_l, t1T_l, zloc, wmod)
    return out_l


def _single_core(xT, x, L, wmod, *, tile, n, in_f, out_f):
    t1T, z = pl.pallas_call(
        _sweep_kernel,
        out_shape=(jax.ShapeDtypeStruct((in_f, n), jnp.float32),
                   jax.ShapeDtypeStruct((in_f, n), jnp.float32)),
        grid=(n // tile,),
        in_specs=[
            pl.BlockSpec((in_f, n), lambda j: (0, 0)),
            pl.BlockSpec((tile, n), lambda j: (j, 0)),
        ],
        out_specs=(
            pl.BlockSpec((in_f, tile), lambda j: (0, j)),
            pl.BlockSpec((in_f, n), lambda j: (0, 0)),
        ),
        compiler_params=pltpu.CompilerParams(
            dimension_semantics=("arbitrary",),
            vmem_limit_bytes=48 * 1024 * 1024),
    )(xT, L)
    zall = jnp.stack([z, jnp.zeros_like(z)])
    return pl.pallas_call(
        _combine_kernel,
        out_shape=jax.ShapeDtypeStruct((n, out_f), jnp.float32),
        grid=(n // tile,),
        in_specs=[
            pl.BlockSpec((tile, in_f), lambda i: (i, 0)),
            pl.BlockSpec((in_f, tile), lambda i: (0, i)),
            pl.BlockSpec((2, in_f, tile), lambda i: (0, 0, i)),
            pl.BlockSpec((3, in_f, out_f), lambda i: (0, 0, 0)),
        ],
        out_specs=pl.BlockSpec((tile, out_f), lambda i: (i, 0)),
        compiler_params=pltpu.CompilerParams(
            dimension_semantics=("arbitrary",),
            vmem_limit_bytes=32 * 1024 * 1024),
    )(x, t1T, zall, wmod)


def kernel(x, L, weight):
    n, in_f = x.shape
    k, _, out_f = weight.shape
    assert k == 3, "kernel specialized for Chebyshev order k=3"
    assert L.shape == (n, n)

    tile = _TILE if n > _TILE else max(n, 8)
    if n % tile:
        n_pad = ((n + tile - 1) // tile) * tile
        x = jnp.zeros((n_pad, in_f), x.dtype).at[:n].set(x)
        L = jnp.zeros((n_pad, n_pad), L.dtype).at[:n, :n].set(L)
    else:
        n_pad = n
    nt = n_pad // tile

    xT = x.T
    wf = weight.astype(jnp.float32)
    wmod = jnp.stack([wf[0] - wf[2], wf[1], wf[2]])

    devs = jax.devices()
    if len(devs) >= 2 and nt % 2 == 0:
        mesh = Mesh(np.array(devs[:2]), ("d",))
        fn = _shard_map(
            functools.partial(_run_local, tile=tile, n=n_pad,
                              in_f=in_f, out_f=out_f),
            mesh=mesh,
            in_specs=(P(None, None), P("d", None), P("d", None),
                      P(None, None, None)),
            out_specs=P("d", None),
            check_vma=False,
        )
        out = fn(xT, x, L, wmod)
    else:
        out = _single_core(xT, x, L, wmod,
                           tile=tile, n=n_pad, in_f=in_f, out_f=out_f)

    return out[:n]


# speedup vs baseline: 13.8090x; 13.8090x over previous
import jax
import jax.numpy as jnp
from jax.experimental import pallas as pl
from jax.experimental.pallas import tpu as pltpu

_TILE = 512


def _dot_tb(a, b):
    return jax.lax.dot_general(a, b, (((1,), (1,)), ((), ())),
                               preferred_element_type=jnp.float32)


def _dot_ta(a, w):
    return jax.lax.dot_general(a, w, (((0,), (0,)), ((), ())),
                               preferred_element_type=jnp.float32)


def _fused_kernel(xT_ref, L_ref, x_ref, w_ref, o_ref, t1T_ref, z_ref,
                  *, tile):
    ph = pl.program_id(0)
    i = pl.program_id(1)

    @pl.when(ph == 0)
    def _sweep():
        y = _dot_tb(xT_ref[...], L_ref[...])
        t1T_ref[:, pl.ds(i * tile, tile)] = y

        @pl.when(i == 0)
        def _init():
            z_ref[...] = jnp.zeros_like(z_ref)

        z_ref[...] += jax.lax.dot_general(
            y, L_ref[...], (((1,), (0,)), ((), ())),
            preferred_element_type=jnp.float32)

    @pl.when(ph == 1)
    def _combine():
        t1b = t1T_ref[:, pl.ds(i * tile, tile)]
        zb = z_ref[:, pl.ds(i * tile, tile)]
        out = jnp.dot(x_ref[...], w_ref[0], preferred_element_type=jnp.float32)
        out += _dot_ta(t1b, w_ref[1])
        out += _dot_ta(zb, w_ref[2])
        o_ref[...] = out


def kernel(x, L, weight):
    n, in_f = x.shape
    k, _, out_f = weight.shape
    assert k == 3, "kernel specialized for Chebyshev order k=3"
    assert L.shape == (n, n)

    tile = _TILE if n > _TILE else max(n, 8)
    if n % tile:
        n_pad = ((n + tile - 1) // tile) * tile
        x = jnp.zeros((n_pad, in_f), x.dtype).at[:n].set(x)
        L = jnp.zeros((n_pad, n_pad), L.dtype).at[:n, :n].set(L)
    else:
        n_pad = n
    nt = n_pad // tile

    xT = x.T
    wf = weight.astype(jnp.float32)
    wmod = jnp.stack([wf[0] - wf[2], wf[1], 2.0 * wf[2]])

    import functools
    out = pl.pallas_call(
        functools.partial(_fused_kernel, tile=tile),
        out_shape=jax.ShapeDtypeStruct((n_pad, out_f), jnp.float32),
        grid=(2, nt),
        in_specs=[
            pl.BlockSpec((in_f, n_pad), lambda ph, i: (0, 0)),
            pl.BlockSpec((tile, n_pad),
                         lambda ph, i: (jnp.where(ph == 0, i, nt - 1), 0)),
            pl.BlockSpec((tile, in_f),
                         lambda ph, i: (jnp.where(ph == 1, i, 0), 0)),
            pl.BlockSpec((3, in_f, out_f), lambda ph, i: (0, 0, 0)),
        ],
        out_specs=pl.BlockSpec((tile, out_f),
                               lambda ph, i: (jnp.where(ph == 1, i, 0), 0)),
        scratch_shapes=[
            pltpu.VMEM((in_f, n_pad), jnp.float32),
            pltpu.VMEM((in_f, n_pad), jnp.float32),
        ],
        compiler_params=pltpu.CompilerParams(
            dimension_semantics=("arbitrary", "arbitrary"),
            vmem_limit_bytes=48 * 1024 * 1024),
    )(xT, L, x, wmod)

    return out[:n]


# speedup vs baseline: 14.4157x; 1.0439x over previous
import jax
import jax.numpy as jnp
from jax.experimental import pallas as pl
from jax.experimental.pallas import tpu as pltpu

_TILE = 1024


def _dot_tb(a, b):
    return jax.lax.dot_general(a, b, (((1,), (1,)), ((), ())),
                               preferred_element_type=jnp.float32)


def _dot_ta(a, w):
    return jax.lax.dot_general(a, w, (((0,), (0,)), ((), ())),
                               preferred_element_type=jnp.float32)


def _fused_kernel(xT_ref, L_ref, x_ref, w_ref, o_ref, t1T_ref, z_ref,
                  *, tile):
    ph = pl.program_id(0)
    i = pl.program_id(1)

    @pl.when(ph == 0)
    def _sweep():
        y = _dot_tb(xT_ref[...], L_ref[...])
        t1T_ref[:, pl.ds(i * tile, tile)] = y

        @pl.when(i == 0)
        def _init():
            z_ref[...] = jnp.zeros_like(z_ref)

        z_ref[...] += jax.lax.dot_general(
            y, L_ref[...], (((1,), (0,)), ((), ())),
            preferred_element_type=jnp.float32)

    @pl.when(ph == 1)
    def _combine():
        t1b = t1T_ref[:, pl.ds(i * tile, tile)]
        zb = z_ref[:, pl.ds(i * tile, tile)]
        out = jnp.dot(x_ref[...], w_ref[0], preferred_element_type=jnp.float32)
        out += _dot_ta(t1b, w_ref[1])
        out += _dot_ta(zb, w_ref[2])
        o_ref[...] = out


def kernel(x, L, weight):
    n, in_f = x.shape
    k, _, out_f = weight.shape
    assert k == 3, "kernel specialized for Chebyshev order k=3"
    assert L.shape == (n, n)

    tile = _TILE if n > _TILE else max(n, 8)
    if n % tile:
        n_pad = ((n + tile - 1) // tile) * tile
        x = jnp.zeros((n_pad, in_f), x.dtype).at[:n].set(x)
        L = jnp.zeros((n_pad, n_pad), L.dtype).at[:n, :n].set(L)
    else:
        n_pad = n
    nt = n_pad // tile

    xT = x.T
    wf = weight.astype(jnp.float32)
    wmod = jnp.stack([wf[0] - wf[2], wf[1], 2.0 * wf[2]])

    import functools
    out = pl.pallas_call(
        functools.partial(_fused_kernel, tile=tile),
        out_shape=jax.ShapeDtypeStruct((n_pad, out_f), jnp.float32),
        grid=(2, nt),
        in_specs=[
            pl.BlockSpec((in_f, n_pad), lambda ph, i: (0, 0)),
            pl.BlockSpec((tile, n_pad),
                         lambda ph, i: (jnp.where(ph == 0, i, nt - 1), 0)),
            pl.BlockSpec((tile, in_f),
                         lambda ph, i: (jnp.where(ph == 1, i, 0), 0)),
            pl.BlockSpec((3, in_f, out_f), lambda ph, i: (0, 0, 0)),
        ],
        out_specs=pl.BlockSpec((tile, out_f),
                               lambda ph, i: (jnp.where(ph == 1, i, 0), 0)),
        scratch_shapes=[
            pltpu.VMEM((in_f, n_pad), jnp.float32),
            pltpu.VMEM((in_f, n_pad), jnp.float32),
        ],
        compiler_params=pltpu.CompilerParams(
            dimension_semantics=("arbitrary", "arbitrary"),
            vmem_limit_bytes=48 * 1024 * 1024),
    )(xT, L, x, wmod)

    return out[:n]
